# Initial kernel scaffold; baseline (speedup 1.0000x reference)
import jax
import jax.numpy as jnp
from jax.experimental import pallas as pl


def kernel(x, edge_index, edge_attr, batch, pre1_W, pre1_b, pre2_W, pre2_b, conv0_W, conv0_b, conv1_W, conv1_b, conv2_W, conv2_b):
    raise NotImplementedError("write your pallas kernel here")


# speedup vs baseline: 11.2766x; 11.2766x over previous
import functools

import jax
import jax.numpy as jnp
from jax import lax
from jax.experimental import pallas as pl
from jax.experimental.pallas import tpu as pltpu
from jax.experimental.pallas import tpu_sc as plsc

N_NODES = 10000
NP = 10240
D = 128
N_EDGES = 320000
EP = 360448
ER = EP // 128
TR = ER // 32
NG = 32
NGP = 128
BR = 12288
STRIPE = NP // 16
R = 1024

_f32 = jnp.float32
_i32 = jnp.int32
_MESH = dict(core_axis_name="c", subcore_axis_name="s")


def _sc_deg_body(row_hbm, ew_hbm, deg_hbm, row_v, ew_v, vbuf, acc_sh):
    c = lax.axis_index("c")
    s = lax.axis_index("s")
    wid = s * 2 + c
    pltpu.sync_copy(row_hbm.at[wid], row_v)
    pltpu.sync_copy(ew_hbm.at[wid], ew_v)
    for t in range(STRIPE // 16):
        vbuf[pl.ds(t * 16, 16)] = jnp.zeros((16,), _f32)
    pltpu.sync_copy(vbuf, acc_sh.at[pl.ds(s * STRIPE, STRIPE)])
    plsc.subcore_barrier()

    def body(j, carry):
        pltpu.sync_copy(ew_v.at[j], acc_sh.at[row_v.at[j]], add=True)
        return carry

    lax.fori_loop(0, TR, body, 0)
    plsc.subcore_barrier()
    pltpu.sync_copy(acc_sh.at[pl.ds(s * STRIPE, STRIPE)], vbuf)
    pltpu.sync_copy(vbuf, deg_hbm.at[pl.ds(c * NP + s * STRIPE, STRIPE)])


@jax.jit
def _sc_deg(row3d, ew3d):
    return pl.kernel(
        _sc_deg_body,
        out_type=jax.ShapeDtypeStruct((2 * NP,), _f32),
        mesh=plsc.VectorSubcoreMesh(**_MESH),
        scratch_types=[
            pltpu.VMEM((TR, 128), _i32),
            pltpu.VMEM((TR, 128), _f32),
            pltpu.VMEM((STRIPE,), _f32),
            pltpu.VMEM_SHARED((NP,), _f32),
        ],
    )(row3d, ew3d)


def _sc_norm_body(row_hbm, col_hbm, ew_hbm, dinv_hbm, norm_hbm,
                  row_v, col_v, ew_v, norm_v, dr_v, dc_v, sem):
    c = lax.axis_index("c")
    s = lax.axis_index("s")
    wid = s * 2 + c
    pltpu.sync_copy(row_hbm.at[wid], row_v)
    pltpu.sync_copy(col_hbm.at[wid], col_v)
    pltpu.sync_copy(ew_hbm.at[wid], ew_v)

    def chunk(j, carry):
        cp1 = pltpu.async_copy(dinv_hbm.at[row_v.at[j]], dr_v, sem)
        cp2 = pltpu.async_copy(dinv_hbm.at[col_v.at[j]], dc_v, sem)
        cp1.wait()
        cp2.wait()
        for k in range(8):
            sl = pl.ds(k * 16, 16)
            norm_v[j, sl] = dr_v[sl] * ew_v[j, sl] * dc_v[sl]
        return carry

    lax.fori_loop(0, TR, chunk, 0)
    pltpu.sync_copy(norm_v, norm_hbm.at[wid])


@jax.jit
def _sc_norm(row3d, col3d, ew3d, dinv1d):
    return pl.kernel(
        _sc_norm_body,
        out_type=jax.ShapeDtypeStruct((32, TR, 128), _f32),
        mesh=plsc.VectorSubcoreMesh(**_MESH),
        scratch_types=[
            pltpu.VMEM((TR, 128), _i32),
            pltpu.VMEM((TR, 128), _i32),
            pltpu.VMEM((TR, 128), _f32),
            pltpu.VMEM((TR, 128), _f32),
            pltpu.VMEM((128,), _f32),
            pltpu.VMEM((128,), _f32),
            pltpu.SemaphoreType.DMA,
        ],
    )(row3d, col3d, ew3d, dinv1d)


def _sc_agg_body(g_hbm, row_hbm, col_hbm, norm_hbm, out_hbm,
                 row_g, col_g, norm_g, rows_v, acc_sh, sem):
    c = lax.axis_index("c")
    s = lax.axis_index("s")
    wid = s * 2 + c

    def zbody(e, carry):
        for q in range(8):
            rows_v[e, pl.ds(q * 16, 16)] = jnp.zeros((16,), _f32)
        return carry

    lax.fori_loop(0, 128, zbody, 0)
    for t in range(STRIPE // 128):
        pltpu.sync_copy(rows_v, acc_sh.at[pl.ds(s * STRIPE + t * 128, 128)])
    plsc.subcore_barrier()

    def group(gg, carry):
        pltpu.sync_copy(row_hbm.at[wid, pl.ds(gg * 8, 8)], row_g)
        pltpu.sync_copy(col_hbm.at[wid, pl.ds(gg * 8, 8)], col_g)
        pltpu.sync_copy(norm_hbm.at[wid, pl.ds(gg * 8, 8)], norm_g)

        def chunk(jj, carry2):
            pltpu.async_copy(g_hbm.at[row_g.at[jj]], rows_v, sem).wait()
            for k in range(8):
                nv = norm_g[jj, pl.ds(k * 16, 16)]
                for t in range(16):
                    sv = jnp.full((16,), nv[t], _f32)
                    e = k * 16 + t
                    for q in range(8):
                        sl = pl.ds(q * 16, 16)
                        rows_v[e, sl] = rows_v[e, sl] * sv
            pltpu.sync_copy(rows_v, acc_sh.at[col_g.at[jj]], add=True)
            return carry2

        lax.fori_loop(0, 8, chunk, 0)
        return carry

    lax.fori_loop(0, TR // 8, group, 0)
    plsc.subcore_barrier()
    for t in range(STRIPE // 128):
        pltpu.sync_copy(acc_sh.at[pl.ds(s * STRIPE + t * 128, 128)], rows_v)
        pltpu.sync_copy(rows_v, out_hbm.at[c, pl.ds(s * STRIPE + t * 128, 128)])


@jax.jit
def _sc_agg(g, row3d, col3d, norm3d):
    return pl.kernel(
        _sc_agg_body,
        out_type=jax.ShapeDtypeStruct((2, NP, D), _f32),
        mesh=plsc.VectorSubcoreMesh(**_MESH),
        scratch_types=[
            pltpu.VMEM((8, 128), _i32),
            pltpu.VMEM((8, 128), _i32),
            pltpu.VMEM((8, 128), _f32),
            pltpu.VMEM((128, D), _f32),
            pltpu.VMEM_SHARED((NP, D), _f32),
            pltpu.SemaphoreType.DMA,
        ],
    )(g, row3d, col3d, norm3d)


def _tc_pre_body(x_ref, degp_ref, w1_ref, b1_ref, w2_ref, b2_ref, w0_ref,
                 g0_ref, dinv_ref):
    h = jnp.maximum(jnp.dot(x_ref[...], w1_ref[...],
                            preferred_element_type=_f32,
                 precision=lax.Precision.HIGHEST) + b1_ref[...], 0.0)
    h = jnp.maximum(jnp.dot(h, w2_ref[...],
                            preferred_element_type=_f32,
                 precision=lax.Precision.HIGHEST) + b2_ref[...], 0.0)
    g0_ref[...] = jnp.dot(h, w0_ref[...], preferred_element_type=_f32,
                 precision=lax.Precision.HIGHEST)
    deg = degp_ref[0] + degp_ref[1]
    dinv_ref[...] = jnp.where(deg > 0.0, lax.rsqrt(deg), 0.0)


@jax.jit
def _tc_pre(x_p, degp, w1, b1, w2, b2, w0):
    nb = NP // R
    return pl.pallas_call(
        _tc_pre_body,
        grid=(nb,),
        in_specs=[
            pl.BlockSpec((R, D), lambda i: (i, 0)),
            pl.BlockSpec((2, R // 128, 128), lambda i: (0, i, 0)),
            pl.BlockSpec((D, D), lambda i: (0, 0)),
            pl.BlockSpec((1, D), lambda i: (0, 0)),
            pl.BlockSpec((D, D), lambda i: (0, 0)),
            pl.BlockSpec((1, D), lambda i: (0, 0)),
            pl.BlockSpec((D, D), lambda i: (0, 0)),
        ],
        out_specs=[
            pl.BlockSpec((R, D), lambda i: (i, 0)),
            pl.BlockSpec((R // 128, 128), lambda i: (i, 0)),
        ],
        out_shape=[
            jax.ShapeDtypeStruct((NP, D), _f32),
            jax.ShapeDtypeStruct((NP // 128, 128), _f32),
        ],
    )(x_p, degp, w1, b1, w2, b2, w0)


def _tc_post_body(part_ref, oht_ref, b_ref, w_ref, gn_ref, pool_ref):
    i = pl.program_id(0)
    h = jnp.maximum(part_ref[0] + part_ref[1] + b_ref[...], 0.0)
    gn_ref[...] = jnp.dot(h, w_ref[...], preferred_element_type=_f32,
                 precision=lax.Precision.HIGHEST)
    oht = oht_ref[...]
    ps = jnp.dot(oht, h, preferred_element_type=_f32,
                 precision=lax.Precision.HIGHEST)
    pc = jnp.dot(oht, jnp.ones((R, D), _f32),
                 preferred_element_type=_f32,
                 precision=lax.Precision.HIGHEST)

    @pl.when(i == 0)
    def _():
        pool_ref[...] = jnp.zeros_like(pool_ref)

    pool_ref[pl.ds(0, NG), :] = pool_ref[pl.ds(0, NG), :] + ps
    pool_ref[pl.ds(2 * NG, NG), :] = pool_ref[pl.ds(2 * NG, NG), :] + pc

    @pl.when(i == pl.num_programs(0) - 1)
    def _():
        cnt = jnp.maximum(pool_ref[pl.ds(2 * NG, NG), :], 1.0)
        pool_ref[pl.ds(NG, NG), :] = pool_ref[pl.ds(0, NG), :] / cnt


@jax.jit
def _tc_post(part, oht, b, w):
    nb = NP // R
    return pl.pallas_call(
        _tc_post_body,
        grid=(nb,),
        in_specs=[
            pl.BlockSpec((2, R, D), lambda i: (0, i, 0)),
            pl.BlockSpec((NG, R), lambda i: (0, i)),
            pl.BlockSpec((1, D), lambda i: (0, 0)),
            pl.BlockSpec((D, D), lambda i: (0, 0)),
        ],
        out_specs=[
            pl.BlockSpec((R, D), lambda i: (i, 0)),
            pl.BlockSpec((3 * NG, D), lambda i: (0, 0)),
        ],
        out_shape=[
            jax.ShapeDtypeStruct((NP, D), _f32),
            jax.ShapeDtypeStruct((3 * NG, D), _f32),
        ],
    )(part, oht, b, w)


def kernel(x, edge_index, edge_attr, batch, pre1_W, pre1_b, pre2_W, pre2_b,
           conv0_W, conv0_b, conv1_W, conv1_b, conv2_W, conv2_b):
    row = edge_index[0]
    col = edge_index[1]
    ew = jnp.ravel(edge_attr).astype(_f32)

    loop_idx = jnp.arange(N_NODES, dtype=_i32)
    padn = EP - N_EDGES - N_NODES
    pad_idx = jnp.arange(padn, dtype=_i32) % N_NODES
    row3d = jnp.concatenate([row, loop_idx, pad_idx]).reshape(32, TR, 128)
    col3d = jnp.concatenate([col, loop_idx, pad_idx]).reshape(32, TR, 128)
    ew3d = jnp.concatenate([ew, jnp.ones((N_NODES,), _f32),
                            jnp.zeros((padn,), _f32)]).reshape(32, TR, 128)
    x_p = jnp.pad(x, ((0, NP - N_NODES), (0, 0)))

    batch_p = jnp.concatenate([batch, jnp.full((NP - N_NODES,), NG, _i32)])
    oht = (batch_p[None, :] ==
           jnp.arange(NG, dtype=_i32)[:, None]).astype(_f32)

    deg_flat = _sc_deg(row3d, ew3d)
    g, dinv2d = _tc_pre(x_p, deg_flat.reshape(2, NP // 128, 128),
                        pre1_W, pre1_b.reshape(1, D),
                        pre2_W, pre2_b.reshape(1, D), conv0_W)

    pools = []
    layer_b = (conv0_b, conv1_b, conv2_b)
    layer_wnext = (conv1_W, conv2_W, conv2_W)
    dinv1d = dinv2d.reshape(NP)
    norm3d = _sc_norm(row3d, col3d, ew3d, dinv1d)
    for l in range(3):
        part = _sc_agg(g, row3d, col3d, norm3d)
        g, pool = _tc_post(part, oht, layer_b[l].reshape(1, D),
                           layer_wnext[l])
        pools.append(pool)

    return jnp.concatenate(
        [jnp.concatenate([p[:NG], p[NG:2 * NG]], axis=1) for p in pools],
        axis=1)


# speedup vs baseline: 11.6288x; 1.0312x over previous
import functools

import jax
import jax.numpy as jnp
from jax import lax
from jax.experimental import pallas as pl
from jax.experimental.pallas import tpu as pltpu
from jax.experimental.pallas import tpu_sc as plsc

N_NODES = 10000
NP = 10240
D = 128
N_EDGES = 320000
EP = 360448
ER = EP // 128
TR = ER // 32
NG = 32
NGP = 128
BR = 12288
STRIPE = NP // 16
R = 1024

_f32 = jnp.float32
_i32 = jnp.int32
_MESH = dict(core_axis_name="c", subcore_axis_name="s")


def _sc_deg_body(row_hbm, ew_hbm, deg_hbm, row_v, ew_v, vbuf, acc_sh):
    c = lax.axis_index("c")
    s = lax.axis_index("s")
    wid = s * 2 + c
    pltpu.sync_copy(row_hbm.at[wid], row_v)
    pltpu.sync_copy(ew_hbm.at[wid], ew_v)
    for t in range(STRIPE // 16):
        vbuf[pl.ds(t * 16, 16)] = jnp.zeros((16,), _f32)
    pltpu.sync_copy(vbuf, acc_sh.at[pl.ds(s * STRIPE, STRIPE)])
    plsc.subcore_barrier()

    def body(j, carry):
        pltpu.sync_copy(ew_v.at[j], acc_sh.at[row_v.at[j]], add=True)
        return carry

    lax.fori_loop(0, TR, body, 0)
    plsc.subcore_barrier()
    pltpu.sync_copy(acc_sh.at[pl.ds(s * STRIPE, STRIPE)], vbuf)
    pltpu.sync_copy(vbuf, deg_hbm.at[pl.ds(c * NP + s * STRIPE, STRIPE)])


@jax.jit
def _sc_deg(row3d, ew3d):
    return pl.kernel(
        _sc_deg_body,
        out_type=jax.ShapeDtypeStruct((2 * NP,), _f32),
        mesh=plsc.VectorSubcoreMesh(**_MESH),
        scratch_types=[
            pltpu.VMEM((TR, 128), _i32),
            pltpu.VMEM((TR, 128), _f32),
            pltpu.VMEM((STRIPE,), _f32),
            pltpu.VMEM_SHARED((NP,), _f32),
        ],
    )(row3d, ew3d)


def _sc_norm_body(row_hbm, col_hbm, ew_hbm, dinv_hbm, norm_hbm,
                  row_v, col_v, ew_v, norm_v, dr_v, dc_v, sem):
    c = lax.axis_index("c")
    s = lax.axis_index("s")
    wid = s * 2 + c
    pltpu.sync_copy(row_hbm.at[wid], row_v)
    pltpu.sync_copy(col_hbm.at[wid], col_v)
    pltpu.sync_copy(ew_hbm.at[wid], ew_v)

    def group(gg, carry):
        for j in range(8):
            pltpu.async_copy(dinv_hbm.at[row_v.at[gg * 8 + j]],
                             dr_v.at[j], sem)
            pltpu.async_copy(dinv_hbm.at[col_v.at[gg * 8 + j]],
                             dc_v.at[j], sem)
        for j in range(8):
            pltpu.make_async_copy(dinv_hbm.at[row_v.at[gg * 8 + j]],
                                  dr_v.at[j], sem).wait()
            pltpu.make_async_copy(dinv_hbm.at[col_v.at[gg * 8 + j]],
                                  dc_v.at[j], sem).wait()
        for j in range(8):
            for k in range(8):
                sl = pl.ds(k * 16, 16)
                norm_v[gg * 8 + j, sl] = (dr_v[j, sl] * ew_v[gg * 8 + j, sl]
                                          * dc_v[j, sl])
        return carry

    lax.fori_loop(0, TR // 8, group, 0)
    pltpu.sync_copy(norm_v, norm_hbm.at[wid])


@jax.jit
def _sc_norm(row3d, col3d, ew3d, dinv1d):
    return pl.kernel(
        _sc_norm_body,
        out_type=jax.ShapeDtypeStruct((32, TR, 128), _f32),
        mesh=plsc.VectorSubcoreMesh(**_MESH),
        scratch_types=[
            pltpu.VMEM((TR, 128), _i32),
            pltpu.VMEM((TR, 128), _i32),
            pltpu.VMEM((TR, 128), _f32),
            pltpu.VMEM((TR, 128), _f32),
            pltpu.VMEM((8, 128), _f32),
            pltpu.VMEM((8, 128), _f32),
            pltpu.SemaphoreType.DMA,
        ],
    )(row3d, col3d, ew3d, dinv1d)


def _scale_chunk(rows, norm_g, jj):
    for k in range(8):
        nv = norm_g[jj, pl.ds(k * 16, 16)]
        for t in range(16):
            sv = jnp.full((16,), nv[t], _f32)
            e = k * 16 + t
            for q in range(8):
                sl = pl.ds(q * 16, 16)
                rows[e, sl] = rows[e, sl] * sv


def _sc_agg_body(g_hbm, row_hbm, col_hbm, norm_hbm, out_hbm,
                 row_g, col_g, norm_g, rows_a, rows_b, acc_sh, sem_a, sem_b):
    c = lax.axis_index("c")
    s = lax.axis_index("s")
    wid = s * 2 + c

    def zbody(e, carry):
        for q in range(8):
            rows_a[e, pl.ds(q * 16, 16)] = jnp.zeros((16,), _f32)
        return carry

    lax.fori_loop(0, 128, zbody, 0)
    for t in range(STRIPE // 128):
        pltpu.sync_copy(rows_a, acc_sh.at[pl.ds(s * STRIPE + t * 128, 128)])
    plsc.subcore_barrier()

    def group(gg, carry):
        pltpu.sync_copy(row_hbm.at[wid, pl.ds(gg * 8, 8)], row_g)
        pltpu.sync_copy(col_hbm.at[wid, pl.ds(gg * 8, 8)], col_g)
        pltpu.sync_copy(norm_hbm.at[wid, pl.ds(gg * 8, 8)], norm_g)
        pltpu.async_copy(g_hbm.at[row_g.at[0]], rows_a, sem_a)
        pltpu.async_copy(g_hbm.at[row_g.at[1]], rows_b, sem_b)

        def pair(p, carry2):
            j0 = 2 * p
            j1 = 2 * p + 1
            pltpu.make_async_copy(g_hbm.at[row_g.at[j0]], rows_a, sem_a).wait()
            _scale_chunk(rows_a, norm_g, j0)
            pltpu.sync_copy(rows_a, acc_sh.at[col_g.at[j0]], add=True)

            @pl.when(p < 3)
            def _():
                pltpu.async_copy(g_hbm.at[row_g.at[j0 + 2]], rows_a, sem_a)

            pltpu.make_async_copy(g_hbm.at[row_g.at[j1]], rows_b, sem_b).wait()
            _scale_chunk(rows_b, norm_g, j1)
            pltpu.sync_copy(rows_b, acc_sh.at[col_g.at[j1]], add=True)

            @pl.when(p < 3)
            def _():
                pltpu.async_copy(g_hbm.at[row_g.at[j1 + 2]], rows_b, sem_b)

            return carry2

        lax.fori_loop(0, 4, pair, 0)
        return carry

    lax.fori_loop(0, TR // 8, group, 0)
    plsc.subcore_barrier()
    for t in range(STRIPE // 128):
        pltpu.sync_copy(acc_sh.at[pl.ds(s * STRIPE + t * 128, 128)], rows_a)
        pltpu.sync_copy(rows_a, out_hbm.at[c, pl.ds(s * STRIPE + t * 128, 128)])


@jax.jit
def _sc_agg(g, row3d, col3d, norm3d):
    return pl.kernel(
        _sc_agg_body,
        out_type=jax.ShapeDtypeStruct((2, NP, D), _f32),
        mesh=plsc.VectorSubcoreMesh(**_MESH),
        scratch_types=[
            pltpu.VMEM((8, 128), _i32),
            pltpu.VMEM((8, 128), _i32),
            pltpu.VMEM((8, 128), _f32),
            pltpu.VMEM((128, D), _f32),
            pltpu.VMEM((128, D), _f32),
            pltpu.VMEM_SHARED((NP, D), _f32),
            pltpu.SemaphoreType.DMA,
            pltpu.SemaphoreType.DMA,
        ],
    )(g, row3d, col3d, norm3d)


def _tc_pre_body(x_ref, degp_ref, w1_ref, b1_ref, w2_ref, b2_ref, w0_ref,
                 g0_ref, dinv_ref):
    h = jnp.maximum(jnp.dot(x_ref[...], w1_ref[...],
                            preferred_element_type=_f32,
                 precision=lax.Precision.HIGHEST) + b1_ref[...], 0.0)
    h = jnp.maximum(jnp.dot(h, w2_ref[...],
                            preferred_element_type=_f32,
                 precision=lax.Precision.HIGHEST) + b2_ref[...], 0.0)
    g0_ref[...] = jnp.dot(h, w0_ref[...], preferred_element_type=_f32,
                 precision=lax.Precision.HIGHEST)
    deg = degp_ref[0] + degp_ref[1]
    dinv_ref[...] = jnp.where(deg > 0.0, lax.rsqrt(deg), 0.0)


@jax.jit
def _tc_pre(x_p, degp, w1, b1, w2, b2, w0):
    nb = NP // R
    return pl.pallas_call(
        _tc_pre_body,
        grid=(nb,),
        in_specs=[
            pl.BlockSpec((R, D), lambda i: (i, 0)),
            pl.BlockSpec((2, R // 128, 128), lambda i: (0, i, 0)),
            pl.BlockSpec((D, D), lambda i: (0, 0)),
            pl.BlockSpec((1, D), lambda i: (0, 0)),
            pl.BlockSpec((D, D), lambda i: (0, 0)),
            pl.BlockSpec((1, D), lambda i: (0, 0)),
            pl.BlockSpec((D, D), lambda i: (0, 0)),
        ],
        out_specs=[
            pl.BlockSpec((R, D), lambda i: (i, 0)),
            pl.BlockSpec((R // 128, 128), lambda i: (i, 0)),
        ],
        out_shape=[
            jax.ShapeDtypeStruct((NP, D), _f32),
            jax.ShapeDtypeStruct((NP // 128, 128), _f32),
        ],
    )(x_p, degp, w1, b1, w2, b2, w0)


def _tc_post_body(part_ref, oht_ref, b_ref, w_ref, gn_ref, pool_ref):
    i = pl.program_id(0)
    h = jnp.maximum(part_ref[0] + part_ref[1] + b_ref[...], 0.0)
    gn_ref[...] = jnp.dot(h, w_ref[...], preferred_element_type=_f32,
                 precision=lax.Precision.HIGHEST)
    oht = oht_ref[...]
    ps = jnp.dot(oht, h, preferred_element_type=_f32,
                 precision=lax.Precision.HIGHEST)
    pc = jnp.dot(oht, jnp.ones((R, D), _f32),
                 preferred_element_type=_f32,
                 precision=lax.Precision.HIGHEST)

    @pl.when(i == 0)
    def _():
        pool_ref[...] = jnp.zeros_like(pool_ref)

    pool_ref[pl.ds(0, NG), :] = pool_ref[pl.ds(0, NG), :] + ps
    pool_ref[pl.ds(2 * NG, NG), :] = pool_ref[pl.ds(2 * NG, NG), :] + pc

    @pl.when(i == pl.num_programs(0) - 1)
    def _():
        cnt = jnp.maximum(pool_ref[pl.ds(2 * NG, NG), :], 1.0)
        pool_ref[pl.ds(NG, NG), :] = pool_ref[pl.ds(0, NG), :] / cnt


@jax.jit
def _tc_post(part, oht, b, w):
    nb = NP // R
    return pl.pallas_call(
        _tc_post_body,
        grid=(nb,),
        in_specs=[
            pl.BlockSpec((2, R, D), lambda i: (0, i, 0)),
            pl.BlockSpec((NG, R), lambda i: (0, i)),
            pl.BlockSpec((1, D), lambda i: (0, 0)),
            pl.BlockSpec((D, D), lambda i: (0, 0)),
        ],
        out_specs=[
            pl.BlockSpec((R, D), lambda i: (i, 0)),
            pl.BlockSpec((3 * NG, D), lambda i: (0, 0)),
        ],
        out_shape=[
            jax.ShapeDtypeStruct((NP, D), _f32),
            jax.ShapeDtypeStruct((3 * NG, D), _f32),
        ],
    )(part, oht, b, w)


def kernel(x, edge_index, edge_attr, batch, pre1_W, pre1_b, pre2_W, pre2_b,
           conv0_W, conv0_b, conv1_W, conv1_b, conv2_W, conv2_b):
    row = edge_index[0]
    col = edge_index[1]
    ew = jnp.ravel(edge_attr).astype(_f32)

    loop_idx = jnp.arange(N_NODES, dtype=_i32)
    padn = EP - N_EDGES - N_NODES
    pad_idx = jnp.arange(padn, dtype=_i32) % N_NODES
    row3d = jnp.concatenate([row, loop_idx, pad_idx]).reshape(32, TR, 128)
    col3d = jnp.concatenate([col, loop_idx, pad_idx]).reshape(32, TR, 128)
    ew3d = jnp.concatenate([ew, jnp.ones((N_NODES,), _f32),
                            jnp.zeros((padn,), _f32)]).reshape(32, TR, 128)
    x_p = jnp.pad(x, ((0, NP - N_NODES), (0, 0)))

    batch_p = jnp.concatenate([batch, jnp.full((NP - N_NODES,), NG, _i32)])
    oht = (batch_p[None, :] ==
           jnp.arange(NG, dtype=_i32)[:, None]).astype(_f32)

    deg_flat = _sc_deg(row3d, ew3d)
    g, dinv2d = _tc_pre(x_p, deg_flat.reshape(2, NP // 128, 128),
                        pre1_W, pre1_b.reshape(1, D),
                        pre2_W, pre2_b.reshape(1, D), conv0_W)

    pools = []
    layer_b = (conv0_b, conv1_b, conv2_b)
    layer_wnext = (conv1_W, conv2_W, conv2_W)
    dinv1d = dinv2d.reshape(NP)
    norm3d = _sc_norm(row3d, col3d, ew3d, dinv1d)
    for l in range(3):
        part = _sc_agg(g, row3d, col3d, norm3d)
        g, pool = _tc_post(part, oht, layer_b[l].reshape(1, D),
                           layer_wnext[l])
        pools.append(pool)

    return jnp.concatenate(
        [jnp.concatenate([p[:NG], p[NG:2 * NG]], axis=1) for p in pools],
        axis=1)


# speedup vs baseline: 11.9027x; 1.0236x over previous
import functools

import jax
import jax.numpy as jnp
from jax import lax
from jax.experimental import pallas as pl
from jax.experimental.pallas import tpu as pltpu
from jax.experimental.pallas import tpu_sc as plsc

N_NODES = 10000
NP = 10240
D = 128
N_EDGES = 320000
EP = 360448
ER = EP // 128
TR = ER // 32
NG = 32
NGP = 128
BR = 12288
STRIPE = NP // 16
R = 1024

_f32 = jnp.float32
_i32 = jnp.int32
_MESH = dict(core_axis_name="c", subcore_axis_name="s")


def _sc_deg_body(row_hbm, ew_hbm, deg_hbm, row_v, ew_v, vbuf, acc_sh):
    c = lax.axis_index("c")
    s = lax.axis_index("s")
    wid = s * 2 + c
    pltpu.sync_copy(row_hbm.at[wid], row_v)
    pltpu.sync_copy(ew_hbm.at[wid], ew_v)
    for t in range(STRIPE // 16):
        vbuf[pl.ds(t * 16, 16)] = jnp.zeros((16,), _f32)
    pltpu.sync_copy(vbuf, acc_sh.at[pl.ds(s * STRIPE, STRIPE)])
    plsc.subcore_barrier()

    def body(j, carry):
        pltpu.sync_copy(ew_v.at[j], acc_sh.at[row_v.at[j]], add=True)
        return carry

    lax.fori_loop(0, TR, body, 0)
    plsc.subcore_barrier()
    pltpu.sync_copy(acc_sh.at[pl.ds(s * STRIPE, STRIPE)], vbuf)
    pltpu.sync_copy(vbuf, deg_hbm.at[pl.ds(c * NP + s * STRIPE, STRIPE)])


@jax.jit
def _sc_deg(row3d, ew3d):
    return pl.kernel(
        _sc_deg_body,
        out_type=jax.ShapeDtypeStruct((2 * NP,), _f32),
        mesh=plsc.VectorSubcoreMesh(**_MESH),
        scratch_types=[
            pltpu.VMEM((TR, 128), _i32),
            pltpu.VMEM((TR, 128), _f32),
            pltpu.VMEM((STRIPE,), _f32),
            pltpu.VMEM_SHARED((NP,), _f32),
        ],
    )(row3d, ew3d)


def _sc_norm_body(row_hbm, col_hbm, ew_hbm, dinv_hbm, norm_hbm,
                  row_v, col_v, ew_v, norm_v, dr_v, dc_v, sem):
    c = lax.axis_index("c")
    s = lax.axis_index("s")
    wid = s * 2 + c
    pltpu.sync_copy(row_hbm.at[wid], row_v)
    pltpu.sync_copy(col_hbm.at[wid], col_v)
    pltpu.sync_copy(ew_hbm.at[wid], ew_v)

    def group(gg, carry):
        for j in range(8):
            pltpu.async_copy(dinv_hbm.at[row_v.at[gg * 8 + j]],
                             dr_v.at[j], sem)
            pltpu.async_copy(dinv_hbm.at[col_v.at[gg * 8 + j]],
                             dc_v.at[j], sem)
        for j in range(8):
            pltpu.make_async_copy(dinv_hbm.at[row_v.at[gg * 8 + j]],
                                  dr_v.at[j], sem).wait()
            pltpu.make_async_copy(dinv_hbm.at[col_v.at[gg * 8 + j]],
                                  dc_v.at[j], sem).wait()
        for j in range(8):
            for k in range(8):
                sl = pl.ds(k * 16, 16)
                norm_v[gg * 8 + j, sl] = (dr_v[j, sl] * ew_v[gg * 8 + j, sl]
                                          * dc_v[j, sl])
        return carry

    lax.fori_loop(0, TR // 8, group, 0)
    pltpu.sync_copy(norm_v, norm_hbm.at[wid])


@jax.jit
def _sc_norm(row3d, col3d, ew3d, dinv1d):
    return pl.kernel(
        _sc_norm_body,
        out_type=jax.ShapeDtypeStruct((32, TR, 128), _f32),
        mesh=plsc.VectorSubcoreMesh(**_MESH),
        scratch_types=[
            pltpu.VMEM((TR, 128), _i32),
            pltpu.VMEM((TR, 128), _i32),
            pltpu.VMEM((TR, 128), _f32),
            pltpu.VMEM((TR, 128), _f32),
            pltpu.VMEM((8, 128), _f32),
            pltpu.VMEM((8, 128), _f32),
            pltpu.SemaphoreType.DMA,
        ],
    )(row3d, col3d, ew3d, dinv1d)


def _scale_chunk(rows, norm_g, jj):
    for k in range(8):
        nv = norm_g[jj, pl.ds(k * 16, 16)]
        for t in range(16):
            sv = jnp.full((16,), nv[t], _f32)
            e = k * 16 + t
            for q in range(8):
                sl = pl.ds(q * 16, 16)
                rows[e, sl] = rows[e, sl] * sv


def _sc_agg_body(g_hbm, row_hbm, col_hbm, norm_hbm, out_hbm,
                 row_g, col_g, norm_g, rows_a, rows_b, acc_sh,
                 sem_ga, sem_gb, sem_sa, sem_sb):
    c = lax.axis_index("c")
    s = lax.axis_index("s")
    wid = s * 2 + c

    def zbody(e, carry):
        for q in range(8):
            rows_a[e, pl.ds(q * 16, 16)] = jnp.zeros((16,), _f32)
        return carry

    lax.fori_loop(0, 128, zbody, 0)
    for t in range(STRIPE // 128):
        pltpu.sync_copy(rows_a, acc_sh.at[pl.ds(s * STRIPE + t * 128, 128)])
    plsc.subcore_barrier()

    def group(gg, carry):
        pltpu.sync_copy(row_hbm.at[wid, pl.ds(gg * 8, 8)], row_g)
        pltpu.sync_copy(col_hbm.at[wid, pl.ds(gg * 8, 8)], col_g)
        pltpu.sync_copy(norm_hbm.at[wid, pl.ds(gg * 8, 8)], norm_g)
        pltpu.async_copy(g_hbm.at[row_g.at[0]], rows_a, sem_ga)
        pltpu.async_copy(g_hbm.at[row_g.at[1]], rows_b, sem_gb)

        def pair(p, carry2):
            j0 = 2 * p
            j1 = 2 * p + 1
            pltpu.make_async_copy(g_hbm.at[row_g.at[j0]], rows_a,
                                  sem_ga).wait()
            _scale_chunk(rows_a, norm_g, j0)
            pltpu.async_copy(rows_a, acc_sh.at[col_g.at[j0]], sem_sa,
                             add=True)
            pltpu.make_async_copy(g_hbm.at[row_g.at[j1]], rows_b,
                                  sem_gb).wait()
            _scale_chunk(rows_b, norm_g, j1)
            pltpu.async_copy(rows_b, acc_sh.at[col_g.at[j1]], sem_sb,
                             add=True)

            @pl.when(p < 3)
            def _():
                pltpu.make_async_copy(rows_a, acc_sh.at[col_g.at[j0]],
                                      sem_sa).wait()
                pltpu.async_copy(g_hbm.at[row_g.at[j0 + 2]], rows_a, sem_ga)
                pltpu.make_async_copy(rows_b, acc_sh.at[col_g.at[j1]],
                                      sem_sb).wait()
                pltpu.async_copy(g_hbm.at[row_g.at[j1 + 2]], rows_b, sem_gb)

            return carry2

        lax.fori_loop(0, 4, pair, 0)
        pltpu.make_async_copy(rows_a, acc_sh.at[col_g.at[6]], sem_sa).wait()
        pltpu.make_async_copy(rows_b, acc_sh.at[col_g.at[7]], sem_sb).wait()
        return carry

    lax.fori_loop(0, TR // 8, group, 0)
    plsc.subcore_barrier()
    for t in range(STRIPE // 128):
        pltpu.sync_copy(acc_sh.at[pl.ds(s * STRIPE + t * 128, 128)], rows_a)
        pltpu.sync_copy(rows_a, out_hbm.at[c, pl.ds(s * STRIPE + t * 128, 128)])


@jax.jit
def _sc_agg(g, row3d, col3d, norm3d):
    return pl.kernel(
        _sc_agg_body,
        out_type=jax.ShapeDtypeStruct((2, NP, D), _f32),
        mesh=plsc.VectorSubcoreMesh(**_MESH),
        scratch_types=[
            pltpu.VMEM((8, 128), _i32),
            pltpu.VMEM((8, 128), _i32),
            pltpu.VMEM((8, 128), _f32),
            pltpu.VMEM((128, D), _f32),
            pltpu.VMEM((128, D), _f32),
            pltpu.VMEM_SHARED((NP, D), _f32),
            pltpu.SemaphoreType.DMA,
            pltpu.SemaphoreType.DMA,
            pltpu.SemaphoreType.DMA,
            pltpu.SemaphoreType.DMA,
        ],
    )(g, row3d, col3d, norm3d)


def _tc_pre_body(x_ref, degp_ref, w1_ref, b1_ref, w2_ref, b2_ref, w0_ref,
                 g0_ref, dinv_ref):
    h = jnp.maximum(jnp.dot(x_ref[...], w1_ref[...],
                            preferred_element_type=_f32,
                 precision=lax.Precision.HIGHEST) + b1_ref[...], 0.0)
    h = jnp.maximum(jnp.dot(h, w2_ref[...],
                            preferred_element_type=_f32,
                 precision=lax.Precision.HIGHEST) + b2_ref[...], 0.0)
    g0_ref[...] = jnp.dot(h, w0_ref[...], preferred_element_type=_f32,
                 precision=lax.Precision.HIGHEST)
    deg = degp_ref[0] + degp_ref[1]
    dinv_ref[...] = jnp.where(deg > 0.0, lax.rsqrt(deg), 0.0)


@jax.jit
def _tc_pre(x_p, degp, w1, b1, w2, b2, w0):
    nb = NP // R
    return pl.pallas_call(
        _tc_pre_body,
        grid=(nb,),
        in_specs=[
            pl.BlockSpec((R, D), lambda i: (i, 0)),
            pl.BlockSpec((2, R // 128, 128), lambda i: (0, i, 0)),
            pl.BlockSpec((D, D), lambda i: (0, 0)),
            pl.BlockSpec((1, D), lambda i: (0, 0)),
            pl.BlockSpec((D, D), lambda i: (0, 0)),
            pl.BlockSpec((1, D), lambda i: (0, 0)),
            pl.BlockSpec((D, D), lambda i: (0, 0)),
        ],
        out_specs=[
            pl.BlockSpec((R, D), lambda i: (i, 0)),
            pl.BlockSpec((R // 128, 128), lambda i: (i, 0)),
        ],
        out_shape=[
            jax.ShapeDtypeStruct((NP, D), _f32),
            jax.ShapeDtypeStruct((NP // 128, 128), _f32),
        ],
    )(x_p, degp, w1, b1, w2, b2, w0)


def _tc_post_body(part_ref, oht_ref, b_ref, w_ref, gn_ref, pool_ref):
    i = pl.program_id(0)
    h = jnp.maximum(part_ref[0] + part_ref[1] + b_ref[...], 0.0)
    gn_ref[...] = jnp.dot(h, w_ref[...], preferred_element_type=_f32,
                 precision=lax.Precision.HIGHEST)
    oht = oht_ref[...]
    ps = jnp.dot(oht, h, preferred_element_type=_f32,
                 precision=lax.Precision.HIGHEST)
    pc = jnp.dot(oht, jnp.ones((R, D), _f32),
                 preferred_element_type=_f32,
                 precision=lax.Precision.HIGHEST)

    @pl.when(i == 0)
    def _():
        pool_ref[...] = jnp.zeros_like(pool_ref)

    pool_ref[pl.ds(0, NG), :] = pool_ref[pl.ds(0, NG), :] + ps
    pool_ref[pl.ds(2 * NG, NG), :] = pool_ref[pl.ds(2 * NG, NG), :] + pc

    @pl.when(i == pl.num_programs(0) - 1)
    def _():
        cnt = jnp.maximum(pool_ref[pl.ds(2 * NG, NG), :], 1.0)
        pool_ref[pl.ds(NG, NG), :] = pool_ref[pl.ds(0, NG), :] / cnt


@jax.jit
def _tc_post(part, oht, b, w):
    nb = NP // R
    return pl.pallas_call(
        _tc_post_body,
        grid=(nb,),
        in_specs=[
            pl.BlockSpec((2, R, D), lambda i: (0, i, 0)),
            pl.BlockSpec((NG, R), lambda i: (0, i)),
            pl.BlockSpec((1, D), lambda i: (0, 0)),
            pl.BlockSpec((D, D), lambda i: (0, 0)),
        ],
        out_specs=[
            pl.BlockSpec((R, D), lambda i: (i, 0)),
            pl.BlockSpec((3 * NG, D), lambda i: (0, 0)),
        ],
        out_shape=[
            jax.ShapeDtypeStruct((NP, D), _f32),
            jax.ShapeDtypeStruct((3 * NG, D), _f32),
        ],
    )(part, oht, b, w)


def kernel(x, edge_index, edge_attr, batch, pre1_W, pre1_b, pre2_W, pre2_b,
           conv0_W, conv0_b, conv1_W, conv1_b, conv2_W, conv2_b):
    row = edge_index[0]
    col = edge_index[1]
    ew = jnp.ravel(edge_attr).astype(_f32)

    loop_idx = jnp.arange(N_NODES, dtype=_i32)
    padn = EP - N_EDGES - N_NODES
    pad_idx = jnp.arange(padn, dtype=_i32) % N_NODES
    row3d = jnp.concatenate([row, loop_idx, pad_idx]).reshape(32, TR, 128)
    col3d = jnp.concatenate([col, loop_idx, pad_idx]).reshape(32, TR, 128)
    ew3d = jnp.concatenate([ew, jnp.ones((N_NODES,), _f32),
                            jnp.zeros((padn,), _f32)]).reshape(32, TR, 128)
    x_p = jnp.pad(x, ((0, NP - N_NODES), (0, 0)))

    batch_p = jnp.concatenate([batch, jnp.full((NP - N_NODES,), NG, _i32)])
    oht = (batch_p[None, :] ==
           jnp.arange(NG, dtype=_i32)[:, None]).astype(_f32)

    deg_flat = _sc_deg(row3d, ew3d)
    g, dinv2d = _tc_pre(x_p, deg_flat.reshape(2, NP // 128, 128),
                        pre1_W, pre1_b.reshape(1, D),
                        pre2_W, pre2_b.reshape(1, D), conv0_W)

    pools = []
    layer_b = (conv0_b, conv1_b, conv2_b)
    layer_wnext = (conv1_W, conv2_W, conv2_W)
    dinv1d = dinv2d.reshape(NP)
    norm3d = _sc_norm(row3d, col3d, ew3d, dinv1d)
    for l in range(3):
        part = _sc_agg(g, row3d, col3d, norm3d)
        g, pool = _tc_post(part, oht, layer_b[l].reshape(1, D),
                           layer_wnext[l])
        pools.append(pool)

    return jnp.concatenate(
        [jnp.concatenate([p[:NG], p[NG:2 * NG]], axis=1) for p in pools],
        axis=1)


# speedup vs baseline: 12.9572x; 1.0886x over previous
import functools

import jax
import jax.numpy as jnp
from jax import lax
from jax.experimental import pallas as pl
from jax.experimental.pallas import tpu as pltpu
from jax.experimental.pallas import tpu_sc as plsc

N_NODES = 10000
NP = 10240
D = 128
N_EDGES = 320000
EP = 360448
ER = EP // 128
TR = ER // 32
NG = 32
NGP = 128
BR = 12288
STRIPE = NP // 16
R = 1024

_f32 = jnp.float32
_i32 = jnp.int32
_MESH = dict(core_axis_name="c", subcore_axis_name="s")


def _rsqrt16(x):
    xh = x * 0.5
    y = jnp.full((16,), 1e-3, _f32)
    for _ in range(24):
        y = y * (1.5 - xh * y * y)
    return y


TSTRIPE = NP // 32


def _sc_prep_body(row_hbm, ew_hbm, dbc_hbm, row_v, ew_v, vbuf, bc_v, acc_sh):
    c = lax.axis_index("c")
    s = lax.axis_index("s")
    wid = s * 2 + c
    pltpu.sync_copy(row_hbm.at[s], row_v.at[0])
    pltpu.sync_copy(row_hbm.at[s + 16], row_v.at[1])
    pltpu.sync_copy(ew_hbm.at[s], ew_v.at[0])
    pltpu.sync_copy(ew_hbm.at[s + 16], ew_v.at[1])
    for t in range(STRIPE // 16):
        vbuf[pl.ds(t * 16, 16)] = jnp.zeros((16,), _f32)
    pltpu.sync_copy(vbuf.at[pl.ds(0, STRIPE)],
                    acc_sh.at[pl.ds(s * STRIPE, STRIPE)])
    plsc.subcore_barrier()

    for h in range(2):
        def body(j, carry, _h=h):
            pltpu.sync_copy(ew_v.at[_h, j], acc_sh.at[row_v.at[_h, j]],
                            add=True)
            return carry

        lax.fori_loop(0, TR, body, 0)
    plsc.subcore_barrier()
    pltpu.sync_copy(acc_sh.at[pl.ds(wid * TSTRIPE, TSTRIPE)],
                    vbuf.at[pl.ds(0, TSTRIPE)])

    def conv(gq, carry):
        dv = _rsqrt16(vbuf[pl.ds(gq * 16, 16)])
        for t in range(16):
            sv = jnp.full((16,), dv[t], _f32)
            for q in range(8):
                bc_v[gq * 16 + t, pl.ds(q * 16, 16)] = sv
        return carry

    lax.fori_loop(0, TSTRIPE // 16, conv, 0)
    pltpu.sync_copy(bc_v, dbc_hbm.at[pl.ds(wid * TSTRIPE, TSTRIPE)])


@jax.jit
def _sc_prep(row3d16, ew3d16):
    return pl.kernel(
        _sc_prep_body,
        out_type=jax.ShapeDtypeStruct((NP, D), _f32),
        mesh=plsc.VectorSubcoreMesh(**_MESH),
        scratch_types=[
            pltpu.VMEM((2, TR, 128), _i32),
            pltpu.VMEM((2, TR, 128), _f32),
            pltpu.VMEM((STRIPE,), _f32),
            pltpu.VMEM((TSTRIPE, D), _f32),
            pltpu.VMEM_SHARED((NP,), _f32),
        ],
    )(row3d16, ew3d16)


def _scale_chunk(rows, norm_g, jj):
    for k in range(8):
        nv = norm_g[jj, pl.ds(k * 16, 16)]
        for t in range(16):
            sv = jnp.full((16,), nv[t], _f32)
            e = k * 16 + t
            for q in range(8):
                sl = pl.ds(q * 16, 16)
                rows[e, sl] = rows[e, sl] * sv


def _sc_agg_body(g_hbm, row_hbm, col_hbm, ew_hbm, out_hbm,
                 row_g, col_g, norm_g, rows_a, rows_b, acc_sh,
                 sem_ga, sem_gb, sem_sa, sem_sb):
    c = lax.axis_index("c")
    s = lax.axis_index("s")
    wid = s * 2 + c

    def zbody(e, carry):
        for q in range(8):
            rows_a[e, pl.ds(q * 16, 16)] = jnp.zeros((16,), _f32)
        return carry

    lax.fori_loop(0, 128, zbody, 0)
    for t in range(STRIPE // 128):
        pltpu.sync_copy(rows_a, acc_sh.at[pl.ds(s * STRIPE + t * 128, 128)])
    plsc.subcore_barrier()

    def group(gg, carry):
        pltpu.sync_copy(row_hbm.at[wid, pl.ds(gg * 8, 8)], row_g)
        pltpu.sync_copy(col_hbm.at[wid, pl.ds(gg * 8, 8)], col_g)
        pltpu.sync_copy(ew_hbm.at[wid, pl.ds(gg * 8, 8)], norm_g)
        pltpu.async_copy(g_hbm.at[row_g.at[0]], rows_a, sem_ga)
        pltpu.async_copy(g_hbm.at[row_g.at[1]], rows_b, sem_gb)

        def pair(p, carry2):
            j0 = 2 * p
            j1 = 2 * p + 1
            pltpu.make_async_copy(g_hbm.at[row_g.at[j0]], rows_a,
                                  sem_ga).wait()
            _scale_chunk(rows_a, norm_g, j0)
            pltpu.async_copy(rows_a, acc_sh.at[col_g.at[j0]], sem_sa,
                             add=True)
            pltpu.make_async_copy(g_hbm.at[row_g.at[j1]], rows_b,
                                  sem_gb).wait()
            _scale_chunk(rows_b, norm_g, j1)
            pltpu.async_copy(rows_b, acc_sh.at[col_g.at[j1]], sem_sb,
                             add=True)

            @pl.when(p < 3)
            def _():
                pltpu.make_async_copy(rows_a, acc_sh.at[col_g.at[j0]],
                                      sem_sa).wait()
                pltpu.async_copy(g_hbm.at[row_g.at[j0 + 2]], rows_a, sem_ga)
                pltpu.make_async_copy(rows_b, acc_sh.at[col_g.at[j1]],
                                      sem_sb).wait()
                pltpu.async_copy(g_hbm.at[row_g.at[j1 + 2]], rows_b, sem_gb)

            return carry2

        lax.fori_loop(0, 4, pair, 0)
        pltpu.make_async_copy(rows_a, acc_sh.at[col_g.at[6]], sem_sa).wait()
        pltpu.make_async_copy(rows_b, acc_sh.at[col_g.at[7]], sem_sb).wait()
        return carry

    lax.fori_loop(0, TR // 8, group, 0)
    plsc.subcore_barrier()
    for t in range(STRIPE // 128):
        pltpu.sync_copy(acc_sh.at[pl.ds(s * STRIPE + t * 128, 128)], rows_a)
        pltpu.sync_copy(rows_a, out_hbm.at[c, pl.ds(s * STRIPE + t * 128, 128)])


@jax.jit
def _sc_agg(g, row3d, col3d, ew3d):
    return pl.kernel(
        _sc_agg_body,
        out_type=jax.ShapeDtypeStruct((2, NP, D), _f32),
        mesh=plsc.VectorSubcoreMesh(**_MESH),
        scratch_types=[
            pltpu.VMEM((8, 128), _i32),
            pltpu.VMEM((8, 128), _i32),
            pltpu.VMEM((8, 128), _f32),
            pltpu.VMEM((128, D), _f32),
            pltpu.VMEM((128, D), _f32),
            pltpu.VMEM_SHARED((NP, D), _f32),
            pltpu.SemaphoreType.DMA,
            pltpu.SemaphoreType.DMA,
            pltpu.SemaphoreType.DMA,
            pltpu.SemaphoreType.DMA,
        ],
    )(g, row3d, col3d, ew3d)


def _tc_pre_body(x_ref, dbc_ref, w1_ref, b1_ref, w2_ref, b2_ref, w0_ref,
                 gs0_ref):
    h = jnp.maximum(jnp.dot(x_ref[...], w1_ref[...],
                            preferred_element_type=_f32,
                 precision=lax.Precision.HIGHEST) + b1_ref[...], 0.0)
    h = jnp.maximum(jnp.dot(h, w2_ref[...],
                            preferred_element_type=_f32,
                 precision=lax.Precision.HIGHEST) + b2_ref[...], 0.0)
    gs0_ref[...] = dbc_ref[...] * jnp.dot(
        h, w0_ref[...], preferred_element_type=_f32,
        precision=lax.Precision.HIGHEST)


@jax.jit
def _tc_pre(x_p, dbc, w1, b1, w2, b2, w0):
    nb = NP // R
    return pl.pallas_call(
        _tc_pre_body,
        grid=(nb,),
        in_specs=[
            pl.BlockSpec((R, D), lambda i: (i, 0)),
            pl.BlockSpec((R, D), lambda i: (i, 0)),
            pl.BlockSpec((D, D), lambda i: (0, 0)),
            pl.BlockSpec((1, D), lambda i: (0, 0)),
            pl.BlockSpec((D, D), lambda i: (0, 0)),
            pl.BlockSpec((1, D), lambda i: (0, 0)),
            pl.BlockSpec((D, D), lambda i: (0, 0)),
        ],
        out_specs=pl.BlockSpec((R, D), lambda i: (i, 0)),
        out_shape=jax.ShapeDtypeStruct((NP, D), _f32),
    )(x_p, dbc, w1, b1, w2, b2, w0)


def _tc_post_body(part_ref, dbc_ref, oht_ref, b_ref, w_ref, gn_ref, pool_ref):
    i = pl.program_id(0)
    h = jnp.maximum(dbc_ref[...] * (part_ref[0] + part_ref[1]) + b_ref[...],
                    0.0)
    gn_ref[...] = dbc_ref[...] * jnp.dot(
        h, w_ref[...], preferred_element_type=_f32,
        precision=lax.Precision.HIGHEST)
    oht = oht_ref[...]
    ps = jnp.dot(oht, h, preferred_element_type=_f32,
                 precision=lax.Precision.HIGHEST)
    pc = jnp.dot(oht, jnp.ones((R, D), _f32),
                 preferred_element_type=_f32,
                 precision=lax.Precision.HIGHEST)

    @pl.when(i == 0)
    def _():
        pool_ref[...] = jnp.zeros_like(pool_ref)

    pool_ref[pl.ds(0, NG), :] = pool_ref[pl.ds(0, NG), :] + ps
    pool_ref[pl.ds(2 * NG, NG), :] = pool_ref[pl.ds(2 * NG, NG), :] + pc

    @pl.when(i == pl.num_programs(0) - 1)
    def _():
        cnt = jnp.maximum(pool_ref[pl.ds(2 * NG, NG), :], 1.0)
        pool_ref[pl.ds(NG, NG), :] = pool_ref[pl.ds(0, NG), :] / cnt


@jax.jit
def _tc_post(part, dbc, oht, b, w):
    nb = NP // R
    return pl.pallas_call(
        _tc_post_body,
        grid=(nb,),
        in_specs=[
            pl.BlockSpec((2, R, D), lambda i: (0, i, 0)),
            pl.BlockSpec((R, D), lambda i: (i, 0)),
            pl.BlockSpec((NG, R), lambda i: (0, i)),
            pl.BlockSpec((1, D), lambda i: (0, 0)),
            pl.BlockSpec((D, D), lambda i: (0, 0)),
        ],
        out_specs=[
            pl.BlockSpec((R, D), lambda i: (i, 0)),
            pl.BlockSpec((3 * NG, D), lambda i: (0, 0)),
        ],
        out_shape=[
            jax.ShapeDtypeStruct((NP, D), _f32),
            jax.ShapeDtypeStruct((3 * NG, D), _f32),
        ],
    )(part, dbc, oht, b, w)


def kernel(x, edge_index, edge_attr, batch, pre1_W, pre1_b, pre2_W, pre2_b,
           conv0_W, conv0_b, conv1_W, conv1_b, conv2_W, conv2_b):
    row = edge_index[0]
    col = edge_index[1]
    ew = jnp.ravel(edge_attr).astype(_f32)

    loop_idx = jnp.arange(N_NODES, dtype=_i32)
    padn = EP - N_EDGES - N_NODES
    pad_idx = jnp.arange(padn, dtype=_i32) % N_NODES
    row3d = jnp.concatenate([row, loop_idx, pad_idx]).reshape(32, TR, 128)
    col3d = jnp.concatenate([col, loop_idx, pad_idx]).reshape(32, TR, 128)
    ew3d = jnp.concatenate([ew, jnp.ones((N_NODES,), _f32),
                            jnp.zeros((padn,), _f32)]).reshape(32, TR, 128)
    x_p = jnp.pad(x, ((0, NP - N_NODES), (0, 0)))

    batch_p = jnp.concatenate([batch, jnp.full((NP - N_NODES,), NG, _i32)])
    oht = (batch_p[None, :] ==
           jnp.arange(NG, dtype=_i32)[:, None]).astype(_f32)

    dbc = _sc_prep(row3d, ew3d)
    gs = _tc_pre(x_p, dbc, pre1_W, pre1_b.reshape(1, D),
                 pre2_W, pre2_b.reshape(1, D), conv0_W)

    pools = []
    layer_b = (conv0_b, conv1_b, conv2_b)
    layer_wnext = (conv1_W, conv2_W, conv2_W)
    for l in range(3):
        part = _sc_agg(gs, row3d, col3d, ew3d)
        gs, pool = _tc_post(part, dbc, oht, layer_b[l].reshape(1, D),
                            layer_wnext[l])
        pools.append(pool)

    return jnp.concatenate(
        [jnp.concatenate([p[:NG], p[NG:2 * NG]], axis=1) for p in pools],
        axis=1)


# speedup vs baseline: 13.3998x; 1.0342x over previous
import functools

import jax
import jax.numpy as jnp
from jax import lax
from jax.experimental import pallas as pl
from jax.experimental.pallas import tpu as pltpu
from jax.experimental.pallas import tpu_sc as plsc

N_NODES = 10000
NP = 10240
D = 128
N_EDGES = 320000
EP = 360448
ER = EP // 128
TR = ER // 32
NG = 32
NGP = 128
BR = 12288
STRIPE = NP // 16
R = 1024

_f32 = jnp.float32
_i32 = jnp.int32
_MESH = dict(core_axis_name="c", subcore_axis_name="s")


def _rsqrt16(x):
    xh = x * 0.5
    y = jnp.full((16,), 1e-3, _f32)
    for _ in range(24):
        y = y * (1.5 - xh * y * y)
    return y


TSTRIPE = NP // 32


def _sc_prep_body(row_hbm, ew_hbm, dbc_hbm, row_v, ew_v, vbuf, bc_v, acc_sh,
                  sem):
    c = lax.axis_index("c")
    s = lax.axis_index("s")
    wid = s * 2 + c
    pltpu.sync_copy(row_hbm.at[s], row_v.at[0])
    pltpu.sync_copy(row_hbm.at[s + 16], row_v.at[1])
    pltpu.sync_copy(ew_hbm.at[s], ew_v.at[0])
    pltpu.sync_copy(ew_hbm.at[s + 16], ew_v.at[1])
    for t in range(STRIPE // 16):
        vbuf[pl.ds(t * 16, 16)] = jnp.zeros((16,), _f32)
    pltpu.sync_copy(vbuf.at[pl.ds(0, STRIPE)],
                    acc_sh.at[pl.ds(s * STRIPE, STRIPE)])
    plsc.subcore_barrier()

    def body(gg, carry):
        for h in range(2):
            for j in range(8):
                pltpu.async_copy(ew_v.at[h, gg * 8 + j],
                                 acc_sh.at[row_v.at[h, gg * 8 + j]], sem,
                                 add=True)
        for h in range(2):
            for j in range(8):
                pltpu.make_async_copy(ew_v.at[h, gg * 8 + j],
                                      acc_sh.at[row_v.at[h, gg * 8 + j]],
                                      sem).wait()
        return carry

    lax.fori_loop(0, TR // 8, body, 0)
    plsc.subcore_barrier()
    pltpu.sync_copy(acc_sh.at[pl.ds(wid * TSTRIPE, TSTRIPE)],
                    vbuf.at[pl.ds(0, TSTRIPE)])

    def conv(gq, carry):
        dv = _rsqrt16(vbuf[pl.ds(gq * 16, 16)])
        for t in range(16):
            sv = jnp.full((16,), dv[t], _f32)
            for q in range(8):
                bc_v[gq * 16 + t, pl.ds(q * 16, 16)] = sv
        return carry

    lax.fori_loop(0, TSTRIPE // 16, conv, 0)
    pltpu.sync_copy(bc_v, dbc_hbm.at[pl.ds(wid * TSTRIPE, TSTRIPE)])


@jax.jit
def _sc_prep(row3d16, ew3d16):
    return pl.kernel(
        _sc_prep_body,
        out_type=jax.ShapeDtypeStruct((NP, D), _f32),
        mesh=plsc.VectorSubcoreMesh(**_MESH),
        scratch_types=[
            pltpu.VMEM((2, TR, 128), _i32),
            pltpu.VMEM((2, TR, 128), _f32),
            pltpu.VMEM((STRIPE,), _f32),
            pltpu.VMEM((TSTRIPE, D), _f32),
            pltpu.VMEM_SHARED((NP,), _f32),
            pltpu.SemaphoreType.DMA,
        ],
    )(row3d16, ew3d16)


def _scale_chunk(rows, norm_g, jj):
    for k in range(8):
        nv = norm_g[jj, pl.ds(k * 16, 16)]
        for t in range(16):
            sv = jnp.full((16,), nv[t], _f32)
            e = k * 16 + t
            for q in range(8):
                sl = pl.ds(q * 16, 16)
                rows[e, sl] = rows[e, sl] * sv


def _sc_agg_body(g_hbm, rc_hbm, ew_hbm, out_hbm,
                 rc_g, norm_g, rows_a, rows_b, acc_sh,
                 sem_ga, sem_gb, sem_sa, sem_sb):
    c = lax.axis_index("c")
    s = lax.axis_index("s")
    wid = s * 2 + c

    def zbody(e, carry):
        for q in range(8):
            rows_a[e, pl.ds(q * 16, 16)] = jnp.zeros((16,), _f32)
        return carry

    lax.fori_loop(0, 128, zbody, 0)
    for t in range(STRIPE // 128):
        pltpu.sync_copy(rows_a, acc_sh.at[pl.ds(s * STRIPE + t * 128, 128)])
    plsc.subcore_barrier()

    def group(gg, carry):
        pltpu.sync_copy(rc_hbm.at[wid, pl.ds(gg * 8, 8)], rc_g)
        pltpu.sync_copy(ew_hbm.at[wid, pl.ds(gg * 8, 8)], norm_g)
        pltpu.async_copy(g_hbm.at[rc_g.at[0, 0]], rows_a, sem_ga)
        pltpu.async_copy(g_hbm.at[rc_g.at[1, 0]], rows_b, sem_gb)

        def pair(p, carry2):
            j0 = 2 * p
            j1 = 2 * p + 1
            pltpu.make_async_copy(g_hbm.at[rc_g.at[j0, 0]], rows_a,
                                  sem_ga).wait()
            _scale_chunk(rows_a, norm_g, j0)
            pltpu.async_copy(rows_a, acc_sh.at[rc_g.at[j0, 1]], sem_sa,
                             add=True)
            pltpu.make_async_copy(g_hbm.at[rc_g.at[j1, 0]], rows_b,
                                  sem_gb).wait()
            _scale_chunk(rows_b, norm_g, j1)
            pltpu.async_copy(rows_b, acc_sh.at[rc_g.at[j1, 1]], sem_sb,
                             add=True)

            @pl.when(p < 3)
            def _():
                pltpu.make_async_copy(rows_a, acc_sh.at[rc_g.at[j0, 1]],
                                      sem_sa).wait()
                pltpu.async_copy(g_hbm.at[rc_g.at[j0 + 2, 0]], rows_a, sem_ga)
                pltpu.make_async_copy(rows_b, acc_sh.at[rc_g.at[j1, 1]],
                                      sem_sb).wait()
                pltpu.async_copy(g_hbm.at[rc_g.at[j1 + 2, 0]], rows_b, sem_gb)

            return carry2

        lax.fori_loop(0, 4, pair, 0)
        pltpu.make_async_copy(rows_a, acc_sh.at[rc_g.at[6, 1]], sem_sa).wait()
        pltpu.make_async_copy(rows_b, acc_sh.at[rc_g.at[7, 1]], sem_sb).wait()
        return carry

    lax.fori_loop(0, TR // 8, group, 0)
    plsc.subcore_barrier()
    for t in range(STRIPE // 128):
        pltpu.sync_copy(acc_sh.at[pl.ds(s * STRIPE + t * 128, 128)],
                        out_hbm.at[c, pl.ds(s * STRIPE + t * 128, 128)])


@jax.jit
def _sc_agg(g, rc4d, ew3d):
    return pl.kernel(
        _sc_agg_body,
        out_type=jax.ShapeDtypeStruct((2, NP, D), _f32),
        mesh=plsc.VectorSubcoreMesh(**_MESH),
        scratch_types=[
            pltpu.VMEM((8, 2, 128), _i32),
            pltpu.VMEM((8, 128), _f32),
            pltpu.VMEM((128, D), _f32),
            pltpu.VMEM((128, D), _f32),
            pltpu.VMEM_SHARED((NP, D), _f32),
            pltpu.SemaphoreType.DMA,
            pltpu.SemaphoreType.DMA,
            pltpu.SemaphoreType.DMA,
            pltpu.SemaphoreType.DMA,
        ],
    )(g, rc4d, ew3d)


def _tc_pre_body(x_ref, dbc_ref, w1_ref, b1_ref, w2_ref, b2_ref, w0_ref,
                 gs0_ref):
    h = jnp.maximum(jnp.dot(x_ref[...], w1_ref[...],
                            preferred_element_type=_f32,
                 precision=lax.Precision.HIGHEST) + b1_ref[...], 0.0)
    h = jnp.maximum(jnp.dot(h, w2_ref[...],
                            preferred_element_type=_f32,
                 precision=lax.Precision.HIGHEST) + b2_ref[...], 0.0)
    gs0_ref[...] = dbc_ref[...] * jnp.dot(
        h, w0_ref[...], preferred_element_type=_f32,
        precision=lax.Precision.HIGHEST)


@jax.jit
def _tc_pre(x_p, dbc, w1, b1, w2, b2, w0):
    nb = NP // R
    return pl.pallas_call(
        _tc_pre_body,
        grid=(nb,),
        in_specs=[
            pl.BlockSpec((R, D), lambda i: (i, 0)),
            pl.BlockSpec((R, D), lambda i: (i, 0)),
            pl.BlockSpec((D, D), lambda i: (0, 0)),
            pl.BlockSpec((1, D), lambda i: (0, 0)),
            pl.BlockSpec((D, D), lambda i: (0, 0)),
            pl.BlockSpec((1, D), lambda i: (0, 0)),
            pl.BlockSpec((D, D), lambda i: (0, 0)),
        ],
        out_specs=pl.BlockSpec((R, D), lambda i: (i, 0)),
        out_shape=jax.ShapeDtypeStruct((NP, D), _f32),
    )(x_p, dbc, w1, b1, w2, b2, w0)


def _tc_post_body(part_ref, dbc_ref, oht_ref, b_ref, w_ref, gn_ref, pool_ref):
    i = pl.program_id(0)
    h = jnp.maximum(dbc_ref[...] * (part_ref[0] + part_ref[1]) + b_ref[...],
                    0.0)
    gn_ref[...] = dbc_ref[...] * jnp.dot(
        h, w_ref[...], preferred_element_type=_f32,
        precision=lax.Precision.HIGHEST)
    oht = oht_ref[...]
    ps = jnp.dot(oht, h, preferred_element_type=_f32,
                 precision=lax.Precision.HIGHEST)
    pc = jnp.dot(oht, jnp.ones((R, D), _f32),
                 preferred_element_type=_f32,
                 precision=lax.Precision.HIGHEST)

    @pl.when(i == 0)
    def _():
        pool_ref[...] = jnp.zeros_like(pool_ref)

    pool_ref[pl.ds(0, NG), :] = pool_ref[pl.ds(0, NG), :] + ps
    pool_ref[pl.ds(2 * NG, NG), :] = pool_ref[pl.ds(2 * NG, NG), :] + pc

    @pl.when(i == pl.num_programs(0) - 1)
    def _():
        cnt = jnp.maximum(pool_ref[pl.ds(2 * NG, NG), :], 1.0)
        pool_ref[pl.ds(NG, NG), :] = pool_ref[pl.ds(0, NG), :] / cnt


@jax.jit
def _tc_post(part, dbc, oht, b, w):
    nb = NP // R
    return pl.pallas_call(
        _tc_post_body,
        grid=(nb,),
        in_specs=[
            pl.BlockSpec((2, R, D), lambda i: (0, i, 0)),
            pl.BlockSpec((R, D), lambda i: (i, 0)),
            pl.BlockSpec((NG, R), lambda i: (0, i)),
            pl.BlockSpec((1, D), lambda i: (0, 0)),
            pl.BlockSpec((D, D), lambda i: (0, 0)),
        ],
        out_specs=[
            pl.BlockSpec((R, D), lambda i: (i, 0)),
            pl.BlockSpec((3 * NG, D), lambda i: (0, 0)),
        ],
        out_shape=[
            jax.ShapeDtypeStruct((NP, D), _f32),
            jax.ShapeDtypeStruct((3 * NG, D), _f32),
        ],
    )(part, dbc, oht, b, w)


def kernel(x, edge_index, edge_attr, batch, pre1_W, pre1_b, pre2_W, pre2_b,
           conv0_W, conv0_b, conv1_W, conv1_b, conv2_W, conv2_b):
    row = edge_index[0]
    col = edge_index[1]
    ew = jnp.ravel(edge_attr).astype(_f32)

    loop_idx = jnp.arange(N_NODES, dtype=_i32)
    padn = EP - N_EDGES - N_NODES
    pad_idx = jnp.arange(padn, dtype=_i32) % N_NODES
    row3d = jnp.concatenate([row, loop_idx, pad_idx]).reshape(32, TR, 128)
    col3d = jnp.concatenate([col, loop_idx, pad_idx]).reshape(32, TR, 128)
    rc4d = jnp.stack([row3d, col3d], axis=2)
    ew3d = jnp.concatenate([ew, jnp.ones((N_NODES,), _f32),
                            jnp.zeros((padn,), _f32)]).reshape(32, TR, 128)
    x_p = jnp.pad(x, ((0, NP - N_NODES), (0, 0)))

    batch_p = jnp.concatenate([batch, jnp.full((NP - N_NODES,), NG, _i32)])
    oht = (batch_p[None, :] ==
           jnp.arange(NG, dtype=_i32)[:, None]).astype(_f32)

    dbc = _sc_prep(row3d, ew3d)
    gs = _tc_pre(x_p, dbc, pre1_W, pre1_b.reshape(1, D),
                 pre2_W, pre2_b.reshape(1, D), conv0_W)

    pools = []
    layer_b = (conv0_b, conv1_b, conv2_b)
    layer_wnext = (conv1_W, conv2_W, conv2_W)
    for l in range(3):
        part = _sc_agg(gs, rc4d, ew3d)
        gs, pool = _tc_post(part, dbc, oht, layer_b[l].reshape(1, D),
                            layer_wnext[l])
        pools.append(pool)

    return jnp.concatenate(
        [jnp.concatenate([p[:NG], p[NG:2 * NG]], axis=1) for p in pools],
        axis=1)


# speedup vs baseline: 13.4978x; 1.0073x over previous
import functools

import jax
import jax.numpy as jnp
from jax import lax
from jax.experimental import pallas as pl
from jax.experimental.pallas import tpu as pltpu
from jax.experimental.pallas import tpu_sc as plsc

N_NODES = 10000
NP = 10240
D = 128
N_EDGES = 320000
EP = 360448
ER = EP // 128
TR = ER // 32
NG = 32
NGP = 128
BR = 12288
STRIPE = NP // 16
R = 1024

_f32 = jnp.float32
_i32 = jnp.int32
_MESH = dict(core_axis_name="c", subcore_axis_name="s")


def _rsqrt16(x):
    xh = x * 0.5
    y = jnp.full((16,), 1e-3, _f32)
    for _ in range(24):
        y = y * (1.5 - xh * y * y)
    return y


TSTRIPE = NP // 32


def _sc_prep_body(row_hbm, ew_hbm, dbc_hbm, row_v, ew_v, vbuf, bc_v, acc_sh,
                  sem):
    c = lax.axis_index("c")
    s = lax.axis_index("s")
    wid = s * 2 + c
    pltpu.sync_copy(row_hbm.at[s], row_v.at[0])
    pltpu.sync_copy(row_hbm.at[s + 16], row_v.at[1])
    pltpu.sync_copy(ew_hbm.at[s], ew_v.at[0])
    pltpu.sync_copy(ew_hbm.at[s + 16], ew_v.at[1])
    for t in range(STRIPE // 16):
        vbuf[pl.ds(t * 16, 16)] = jnp.zeros((16,), _f32)
    pltpu.sync_copy(vbuf.at[pl.ds(0, STRIPE)],
                    acc_sh.at[pl.ds(s * STRIPE, STRIPE)])
    plsc.subcore_barrier()

    def body(gg, carry):
        for h in range(2):
            for j in range(8):
                pltpu.async_copy(ew_v.at[h, gg * 8 + j],
                                 acc_sh.at[row_v.at[h, gg * 8 + j]], sem,
                                 add=True)
        for h in range(2):
            for j in range(8):
                pltpu.make_async_copy(ew_v.at[h, gg * 8 + j],
                                      acc_sh.at[row_v.at[h, gg * 8 + j]],
                                      sem).wait()
        return carry

    lax.fori_loop(0, TR // 8, body, 0)
    plsc.subcore_barrier()
    pltpu.sync_copy(acc_sh.at[pl.ds(wid * TSTRIPE, TSTRIPE)],
                    vbuf.at[pl.ds(0, TSTRIPE)])

    def conv(gq, carry):
        dv = _rsqrt16(vbuf[pl.ds(gq * 16, 16)])
        for t in range(16):
            sv = jnp.full((16,), dv[t], _f32)
            for q in range(8):
                bc_v[gq * 16 + t, pl.ds(q * 16, 16)] = sv
        return carry

    lax.fori_loop(0, TSTRIPE // 16, conv, 0)
    pltpu.sync_copy(bc_v, dbc_hbm.at[pl.ds(wid * TSTRIPE, TSTRIPE)])


@jax.jit
def _sc_prep(row3d16, ew3d16):
    return pl.kernel(
        _sc_prep_body,
        out_type=jax.ShapeDtypeStruct((NP, D), _f32),
        mesh=plsc.VectorSubcoreMesh(**_MESH),
        scratch_types=[
            pltpu.VMEM((2, TR, 128), _i32),
            pltpu.VMEM((2, TR, 128), _f32),
            pltpu.VMEM((STRIPE,), _f32),
            pltpu.VMEM((TSTRIPE, D), _f32),
            pltpu.VMEM_SHARED((NP,), _f32),
            pltpu.SemaphoreType.DMA,
        ],
    )(row3d16, ew3d16)


def _scale_chunk(rows, norm_g, jj):
    for k in range(8):
        nv = norm_g[jj, pl.ds(k * 16, 16)]
        for t in range(16):
            sv = jnp.full((16,), nv[t], _f32)
            e = k * 16 + t
            for q in range(8):
                sl = pl.ds(q * 16, 16)
                rows[e, sl] = rows[e, sl] * sv


def _sc_agg_body(g_hbm, rc_hbm, ew_hbm, out_hbm,
                 rc_g, norm_g, rows_a, rows_b, acc_sh,
                 sem_ga, sem_gb, sem_sa, sem_sb, sem_ix):
    c = lax.axis_index("c")
    s = lax.axis_index("s")
    wid = s * 2 + c
    ngrp = TR // 8

    def zbody(e, carry):
        for q in range(8):
            rows_a[e, pl.ds(q * 16, 16)] = jnp.zeros((16,), _f32)
        return carry

    lax.fori_loop(0, 128, zbody, 0)
    for t in range(STRIPE // 128):
        pltpu.sync_copy(rows_a, acc_sh.at[pl.ds(s * STRIPE + t * 128, 128)])
    plsc.subcore_barrier()

    pltpu.async_copy(rc_hbm.at[wid, pl.ds(0, 8)], rc_g.at[0], sem_ix)
    pltpu.async_copy(ew_hbm.at[wid, pl.ds(0, 8)], norm_g.at[0], sem_ix)
    pltpu.make_async_copy(rc_hbm.at[wid, pl.ds(0, 8)], rc_g.at[0],
                          sem_ix).wait()
    pltpu.make_async_copy(ew_hbm.at[wid, pl.ds(0, 8)], norm_g.at[0],
                          sem_ix).wait()
    pltpu.async_copy(g_hbm.at[rc_g.at[0, 0, 0]], rows_a, sem_ga)
    pltpu.async_copy(g_hbm.at[rc_g.at[0, 1, 0]], rows_b, sem_gb)

    def group(gg, carry):
        par = gg % 2
        nxt = (gg + 1) % 2

        @pl.when(gg < ngrp - 1)
        def _():
            pltpu.async_copy(rc_hbm.at[wid, pl.ds(gg * 8 + 8, 8)],
                             rc_g.at[nxt], sem_ix)
            pltpu.async_copy(ew_hbm.at[wid, pl.ds(gg * 8 + 8, 8)],
                             norm_g.at[nxt], sem_ix)

        def pair(p, carry2):
            j0 = 2 * p
            j1 = 2 * p + 1
            pltpu.make_async_copy(g_hbm.at[rc_g.at[par, j0, 0]], rows_a,
                                  sem_ga).wait()
            _scale_chunk(rows_a, norm_g.at[par], j0)
            pltpu.async_copy(rows_a, acc_sh.at[rc_g.at[par, j0, 1]], sem_sa,
                             add=True)
            pltpu.make_async_copy(g_hbm.at[rc_g.at[par, j1, 0]], rows_b,
                                  sem_gb).wait()
            _scale_chunk(rows_b, norm_g.at[par], j1)
            pltpu.async_copy(rows_b, acc_sh.at[rc_g.at[par, j1, 1]], sem_sb,
                             add=True)

            @pl.when(p < 3)
            def _():
                pltpu.make_async_copy(rows_a, acc_sh.at[rc_g.at[par, j0, 1]],
                                      sem_sa).wait()
                pltpu.async_copy(g_hbm.at[rc_g.at[par, j0 + 2, 0]], rows_a,
                                 sem_ga)
                pltpu.make_async_copy(rows_b, acc_sh.at[rc_g.at[par, j1, 1]],
                                      sem_sb).wait()
                pltpu.async_copy(g_hbm.at[rc_g.at[par, j1 + 2, 0]], rows_b,
                                 sem_gb)

            return carry2

        lax.fori_loop(0, 4, pair, 0)
        pltpu.make_async_copy(rows_a, acc_sh.at[rc_g.at[par, 6, 1]],
                              sem_sa).wait()
        pltpu.make_async_copy(rows_b, acc_sh.at[rc_g.at[par, 7, 1]],
                              sem_sb).wait()

        @pl.when(gg < ngrp - 1)
        def _():
            pltpu.make_async_copy(rc_hbm.at[wid, pl.ds(gg * 8 + 8, 8)],
                                  rc_g.at[nxt], sem_ix).wait()
            pltpu.make_async_copy(ew_hbm.at[wid, pl.ds(gg * 8 + 8, 8)],
                                  norm_g.at[nxt], sem_ix).wait()
            pltpu.async_copy(g_hbm.at[rc_g.at[nxt, 0, 0]], rows_a, sem_ga)
            pltpu.async_copy(g_hbm.at[rc_g.at[nxt, 1, 0]], rows_b, sem_gb)

        return carry

    lax.fori_loop(0, ngrp, group, 0)
    plsc.subcore_barrier()
    for t in range(STRIPE // 128):
        pltpu.sync_copy(acc_sh.at[pl.ds(s * STRIPE + t * 128, 128)],
                        out_hbm.at[c, pl.ds(s * STRIPE + t * 128, 128)])


@jax.jit
def _sc_agg(g, rc4d, ew3d):
    return pl.kernel(
        _sc_agg_body,
        out_type=jax.ShapeDtypeStruct((2, NP, D), _f32),
        mesh=plsc.VectorSubcoreMesh(**_MESH),
        scratch_types=[
            pltpu.VMEM((2, 8, 2, 128), _i32),
            pltpu.VMEM((2, 8, 128), _f32),
            pltpu.VMEM((128, D), _f32),
            pltpu.VMEM((128, D), _f32),
            pltpu.VMEM_SHARED((NP, D), _f32),
            pltpu.SemaphoreType.DMA,
            pltpu.SemaphoreType.DMA,
            pltpu.SemaphoreType.DMA,
            pltpu.SemaphoreType.DMA,
            pltpu.SemaphoreType.DMA,
        ],
    )(g, rc4d, ew3d)


def _tc_pre_body(x_ref, dbc_ref, w1_ref, b1_ref, w2_ref, b2_ref, w0_ref,
                 gs0_ref):
    h = jnp.maximum(jnp.dot(x_ref[...], w1_ref[...],
                            preferred_element_type=_f32,
                 precision=lax.Precision.HIGHEST) + b1_ref[...], 0.0)
    h = jnp.maximum(jnp.dot(h, w2_ref[...],
                            preferred_element_type=_f32,
                 precision=lax.Precision.HIGHEST) + b2_ref[...], 0.0)
    gs0_ref[...] = dbc_ref[...] * jnp.dot(
        h, w0_ref[...], preferred_element_type=_f32,
        precision=lax.Precision.HIGHEST)


@jax.jit
def _tc_pre(x_p, dbc, w1, b1, w2, b2, w0):
    nb = NP // R
    return pl.pallas_call(
        _tc_pre_body,
        grid=(nb,),
        in_specs=[
            pl.BlockSpec((R, D), lambda i: (i, 0)),
            pl.BlockSpec((R, D), lambda i: (i, 0)),
            pl.BlockSpec((D, D), lambda i: (0, 0)),
            pl.BlockSpec((1, D), lambda i: (0, 0)),
            pl.BlockSpec((D, D), lambda i: (0, 0)),
            pl.BlockSpec((1, D), lambda i: (0, 0)),
            pl.BlockSpec((D, D), lambda i: (0, 0)),
        ],
        out_specs=pl.BlockSpec((R, D), lambda i: (i, 0)),
        out_shape=jax.ShapeDtypeStruct((NP, D), _f32),
    )(x_p, dbc, w1, b1, w2, b2, w0)


def _tc_post_body(part_ref, dbc_ref, oht_ref, b_ref, w_ref, gn_ref, pool_ref):
    i = pl.program_id(0)
    h = jnp.maximum(dbc_ref[...] * (part_ref[0] + part_ref[1]) + b_ref[...],
                    0.0)
    gn_ref[...] = dbc_ref[...] * jnp.dot(
        h, w_ref[...], preferred_element_type=_f32,
        precision=lax.Precision.HIGHEST)
    oht = oht_ref[...]
    ps = jnp.dot(oht, h, preferred_element_type=_f32,
                 precision=lax.Precision.HIGHEST)
    pc = jnp.dot(oht, jnp.ones((R, D), _f32),
                 preferred_element_type=_f32,
                 precision=lax.Precision.HIGHEST)

    @pl.when(i == 0)
    def _():
        pool_ref[...] = jnp.zeros_like(pool_ref)

    pool_ref[pl.ds(0, NG), :] = pool_ref[pl.ds(0, NG), :] + ps
    pool_ref[pl.ds(2 * NG, NG), :] = pool_ref[pl.ds(2 * NG, NG), :] + pc

    @pl.when(i == pl.num_programs(0) - 1)
    def _():
        cnt = jnp.maximum(pool_ref[pl.ds(2 * NG, NG), :], 1.0)
        pool_ref[pl.ds(NG, NG), :] = pool_ref[pl.ds(0, NG), :] / cnt


@jax.jit
def _tc_post(part, dbc, oht, b, w):
    nb = NP // R
    return pl.pallas_call(
        _tc_post_body,
        grid=(nb,),
        in_specs=[
            pl.BlockSpec((2, R, D), lambda i: (0, i, 0)),
            pl.BlockSpec((R, D), lambda i: (i, 0)),
            pl.BlockSpec((NG, R), lambda i: (0, i)),
            pl.BlockSpec((1, D), lambda i: (0, 0)),
            pl.BlockSpec((D, D), lambda i: (0, 0)),
        ],
        out_specs=[
            pl.BlockSpec((R, D), lambda i: (i, 0)),
            pl.BlockSpec((3 * NG, D), lambda i: (0, 0)),
        ],
        out_shape=[
            jax.ShapeDtypeStruct((NP, D), _f32),
            jax.ShapeDtypeStruct((3 * NG, D), _f32),
        ],
    )(part, dbc, oht, b, w)


def kernel(x, edge_index, edge_attr, batch, pre1_W, pre1_b, pre2_W, pre2_b,
           conv0_W, conv0_b, conv1_W, conv1_b, conv2_W, conv2_b):
    row = edge_index[0]
    col = edge_index[1]
    ew = jnp.ravel(edge_attr).astype(_f32)

    loop_idx = jnp.arange(N_NODES, dtype=_i32)
    padn = EP - N_EDGES - N_NODES
    pad_idx = jnp.arange(padn, dtype=_i32) % N_NODES
    row3d = jnp.concatenate([row, loop_idx, pad_idx]).reshape(32, TR, 128)
    col3d = jnp.concatenate([col, loop_idx, pad_idx]).reshape(32, TR, 128)
    rc4d = jnp.stack([row3d, col3d], axis=2)
    ew3d = jnp.concatenate([ew, jnp.ones((N_NODES,), _f32),
                            jnp.zeros((padn,), _f32)]).reshape(32, TR, 128)
    x_p = jnp.pad(x, ((0, NP - N_NODES), (0, 0)))

    batch_p = jnp.concatenate([batch, jnp.full((NP - N_NODES,), NG, _i32)])
    oht = (batch_p[None, :] ==
           jnp.arange(NG, dtype=_i32)[:, None]).astype(_f32)

    dbc = _sc_prep(row3d, ew3d)
    gs = _tc_pre(x_p, dbc, pre1_W, pre1_b.reshape(1, D),
                 pre2_W, pre2_b.reshape(1, D), conv0_W)

    pools = []
    layer_b = (conv0_b, conv1_b, conv2_b)
    layer_wnext = (conv1_W, conv2_W, conv2_W)
    for l in range(3):
        part = _sc_agg(gs, rc4d, ew3d)
        gs, pool = _tc_post(part, dbc, oht, layer_b[l].reshape(1, D),
                            layer_wnext[l])
        pools.append(pool)

    return jnp.concatenate(
        [jnp.concatenate([p[:NG], p[NG:2 * NG]], axis=1) for p in pools],
        axis=1)


# speedup vs baseline: 13.5362x; 1.0028x over previous
import jax
import jax.numpy as jnp
from jax import lax
from jax.experimental import pallas as pl
from jax.experimental.pallas import tpu as pltpu
from jax.experimental.pallas import tpu_sc as plsc

N_NODES = 10000
NP = 10240
D = 128
N_EDGES = 320000
EP = 360448
ER = EP // 128
TR = ER // 32
NG = 32
STRIPE = NP // 16
R = 1024

_f32 = jnp.float32
_i32 = jnp.int32
_MESH = dict(core_axis_name="c", subcore_axis_name="s")


def _rsqrt16(x):
    xh = x * 0.5
    y = jnp.full((16,), 1e-3, _f32)
    for _ in range(24):
        y = y * (1.5 - xh * y * y)
    return y


TSTRIPE = NP // 32


def _sc_prep_body(row_hbm, ew_hbm, dbc_hbm, row_v, ew_v, vbuf, bc_v, acc_sh,
                  sem):
    c = lax.axis_index("c")
    s = lax.axis_index("s")
    wid = s * 2 + c
    pltpu.sync_copy(row_hbm.at[s], row_v.at[0])
    pltpu.sync_copy(row_hbm.at[s + 16], row_v.at[1])
    pltpu.sync_copy(ew_hbm.at[s], ew_v.at[0])
    pltpu.sync_copy(ew_hbm.at[s + 16], ew_v.at[1])
    for t in range(STRIPE // 16):
        vbuf[pl.ds(t * 16, 16)] = jnp.zeros((16,), _f32)
    pltpu.sync_copy(vbuf.at[pl.ds(0, STRIPE)],
                    acc_sh.at[pl.ds(s * STRIPE, STRIPE)])
    plsc.subcore_barrier()

    def body(gg, carry):
        for h in range(2):
            for j in range(8):
                pltpu.async_copy(ew_v.at[h, gg * 8 + j],
                                 acc_sh.at[row_v.at[h, gg * 8 + j]], sem,
                                 add=True)
        for h in range(2):
            for j in range(8):
                pltpu.make_async_copy(ew_v.at[h, gg * 8 + j],
                                      acc_sh.at[row_v.at[h, gg * 8 + j]],
                                      sem).wait()
        return carry

    lax.fori_loop(0, TR // 8, body, 0)
    plsc.subcore_barrier()
    pltpu.sync_copy(acc_sh.at[pl.ds(wid * TSTRIPE, TSTRIPE)],
                    vbuf.at[pl.ds(0, TSTRIPE)])

    def conv(gq, carry):
        dv = _rsqrt16(vbuf[pl.ds(gq * 16, 16)])
        for t in range(16):
            sv = jnp.full((16,), dv[t], _f32)
            for q in range(8):
                bc_v[gq * 16 + t, pl.ds(q * 16, 16)] = sv
        return carry

    lax.fori_loop(0, TSTRIPE // 16, conv, 0)
    pltpu.sync_copy(bc_v, dbc_hbm.at[pl.ds(wid * TSTRIPE, TSTRIPE)])


@jax.jit
def _sc_prep(row3d16, ew3d16):
    return pl.kernel(
        _sc_prep_body,
        out_type=jax.ShapeDtypeStruct((NP, D), _f32),
        mesh=plsc.VectorSubcoreMesh(**_MESH),
        scratch_types=[
            pltpu.VMEM((2, TR, 128), _i32),
            pltpu.VMEM((2, TR, 128), _f32),
            pltpu.VMEM((STRIPE,), _f32),
            pltpu.VMEM((TSTRIPE, D), _f32),
            pltpu.VMEM_SHARED((NP,), _f32),
            pltpu.SemaphoreType.DMA,
        ],
    )(row3d16, ew3d16)


def _scale_chunk(rows, norm_g, jj):
    for k in range(8):
        nv = norm_g[jj, pl.ds(k * 16, 16)]
        for t in range(16):
            sv = jnp.full((16,), nv[t], _f32)
            e = k * 16 + t
            for q in range(8):
                sl = pl.ds(q * 16, 16)
                rows[e, sl] = rows[e, sl] * sv


def _sc_agg_body(g_hbm, rc_hbm, ew_hbm, out_hbm,
                 rc_g, norm_g, rows_a, rows_b, acc_sh,
                 sem_ga, sem_gb, sem_sa, sem_sb, sem_ix):
    c = lax.axis_index("c")
    s = lax.axis_index("s")
    wid = s * 2 + c
    ngrp = TR // 8

    def zbody(e, carry):
        for q in range(8):
            rows_a[e, pl.ds(q * 16, 16)] = jnp.zeros((16,), _f32)
        return carry

    lax.fori_loop(0, 128, zbody, 0)
    for t in range(STRIPE // 128):
        pltpu.sync_copy(rows_a, acc_sh.at[pl.ds(s * STRIPE + t * 128, 128)])
    plsc.subcore_barrier()

    pltpu.async_copy(rc_hbm.at[wid, pl.ds(0, 8)], rc_g.at[0], sem_ix)
    pltpu.async_copy(ew_hbm.at[wid, pl.ds(0, 8)], norm_g.at[0], sem_ix)
    pltpu.make_async_copy(rc_hbm.at[wid, pl.ds(0, 8)], rc_g.at[0],
                          sem_ix).wait()
    pltpu.make_async_copy(ew_hbm.at[wid, pl.ds(0, 8)], norm_g.at[0],
                          sem_ix).wait()
    pltpu.async_copy(g_hbm.at[rc_g.at[0, 0, 0]], rows_a, sem_ga)
    pltpu.async_copy(g_hbm.at[rc_g.at[0, 1, 0]], rows_b, sem_gb)

    def group(gg, carry):
        par = gg % 2
        nxt = (gg + 1) % 2

        @pl.when(gg < ngrp - 1)
        def _():
            pltpu.async_copy(rc_hbm.at[wid, pl.ds(gg * 8 + 8, 8)],
                             rc_g.at[nxt], sem_ix)
            pltpu.async_copy(ew_hbm.at[wid, pl.ds(gg * 8 + 8, 8)],
                             norm_g.at[nxt], sem_ix)

        def pair(p, carry2):
            j0 = 2 * p
            j1 = 2 * p + 1
            pltpu.make_async_copy(g_hbm.at[rc_g.at[par, j0, 0]], rows_a,
                                  sem_ga).wait()
            _scale_chunk(rows_a, norm_g.at[par], j0)
            pltpu.async_copy(rows_a, acc_sh.at[rc_g.at[par, j0, 1]], sem_sa,
                             add=True)
            pltpu.make_async_copy(g_hbm.at[rc_g.at[par, j1, 0]], rows_b,
                                  sem_gb).wait()
            _scale_chunk(rows_b, norm_g.at[par], j1)
            pltpu.async_copy(rows_b, acc_sh.at[rc_g.at[par, j1, 1]], sem_sb,
                             add=True)

            @pl.when(p < 3)
            def _():
                pltpu.make_async_copy(rows_a, acc_sh.at[rc_g.at[par, j0, 1]],
                                      sem_sa).wait()
                pltpu.async_copy(g_hbm.at[rc_g.at[par, j0 + 2, 0]], rows_a,
                                 sem_ga)
                pltpu.make_async_copy(rows_b, acc_sh.at[rc_g.at[par, j1, 1]],
                                      sem_sb).wait()
                pltpu.async_copy(g_hbm.at[rc_g.at[par, j1 + 2, 0]], rows_b,
                                 sem_gb)

            return carry2

        lax.fori_loop(0, 4, pair, 0)
        pltpu.make_async_copy(rows_a, acc_sh.at[rc_g.at[par, 6, 1]],
                              sem_sa).wait()
        pltpu.make_async_copy(rows_b, acc_sh.at[rc_g.at[par, 7, 1]],
                              sem_sb).wait()

        @pl.when(gg < ngrp - 1)
        def _():
            pltpu.make_async_copy(rc_hbm.at[wid, pl.ds(gg * 8 + 8, 8)],
                                  rc_g.at[nxt], sem_ix).wait()
            pltpu.make_async_copy(ew_hbm.at[wid, pl.ds(gg * 8 + 8, 8)],
                                  norm_g.at[nxt], sem_ix).wait()
            pltpu.async_copy(g_hbm.at[rc_g.at[nxt, 0, 0]], rows_a, sem_ga)
            pltpu.async_copy(g_hbm.at[rc_g.at[nxt, 1, 0]], rows_b, sem_gb)

        return carry

    lax.fori_loop(0, ngrp, group, 0)
    plsc.subcore_barrier()
    for t in range(STRIPE // 128):
        pltpu.sync_copy(acc_sh.at[pl.ds(s * STRIPE + t * 128, 128)],
                        out_hbm.at[c, pl.ds(s * STRIPE + t * 128, 128)])


@jax.jit
def _sc_agg(g, rc4d, ew3d):
    return pl.kernel(
        _sc_agg_body,
        out_type=jax.ShapeDtypeStruct((2, NP, D), _f32),
        mesh=plsc.VectorSubcoreMesh(**_MESH),
        scratch_types=[
            pltpu.VMEM((2, 8, 2, 128), _i32),
            pltpu.VMEM((2, 8, 128), _f32),
            pltpu.VMEM((128, D), _f32),
            pltpu.VMEM((128, D), _f32),
            pltpu.VMEM_SHARED((NP, D), _f32),
            pltpu.SemaphoreType.DMA,
            pltpu.SemaphoreType.DMA,
            pltpu.SemaphoreType.DMA,
            pltpu.SemaphoreType.DMA,
            pltpu.SemaphoreType.DMA,
        ],
    )(g, rc4d, ew3d)


def _tc_pre_body(x_ref, dbc_ref, w1_ref, b1_ref, w2_ref, b2_ref, w0_ref,
                 gs0_ref):
    h = jnp.maximum(jnp.dot(x_ref[...], w1_ref[...],
                            preferred_element_type=_f32,
                 precision=lax.Precision.HIGHEST) + b1_ref[...], 0.0)
    h = jnp.maximum(jnp.dot(h, w2_ref[...],
                            preferred_element_type=_f32,
                 precision=lax.Precision.HIGHEST) + b2_ref[...], 0.0)
    gs0_ref[...] = dbc_ref[...] * jnp.dot(
        h, w0_ref[...], preferred_element_type=_f32,
        precision=lax.Precision.HIGHEST)


@jax.jit
def _tc_pre(x_p, dbc, w1, b1, w2, b2, w0):
    nb = NP // R
    return pl.pallas_call(
        _tc_pre_body,
        grid=(nb,),
        in_specs=[
            pl.BlockSpec((R, D), lambda i: (i, 0)),
            pl.BlockSpec((R, D), lambda i: (i, 0)),
            pl.BlockSpec((D, D), lambda i: (0, 0)),
            pl.BlockSpec((1, D), lambda i: (0, 0)),
            pl.BlockSpec((D, D), lambda i: (0, 0)),
            pl.BlockSpec((1, D), lambda i: (0, 0)),
            pl.BlockSpec((D, D), lambda i: (0, 0)),
        ],
        out_specs=pl.BlockSpec((R, D), lambda i: (i, 0)),
        out_shape=jax.ShapeDtypeStruct((NP, D), _f32),
    )(x_p, dbc, w1, b1, w2, b2, w0)


def _tc_post_body(part_ref, dbc_ref, oht_ref, b_ref, w_ref, gn_ref, pool_ref):
    i = pl.program_id(0)
    h = jnp.maximum(dbc_ref[...] * (part_ref[0] + part_ref[1]) + b_ref[...],
                    0.0)
    gn_ref[...] = dbc_ref[...] * jnp.dot(
        h, w_ref[...], preferred_element_type=_f32,
        precision=lax.Precision.HIGHEST)
    oht = oht_ref[...]
    ps = jnp.dot(oht, h, preferred_element_type=_f32,
                 precision=lax.Precision.HIGHEST)
    pc = jnp.dot(oht, jnp.ones((R, D), _f32),
                 preferred_element_type=_f32,
                 precision=lax.Precision.HIGHEST)

    @pl.when(i == 0)
    def _():
        pool_ref[...] = jnp.zeros_like(pool_ref)

    pool_ref[pl.ds(0, NG), :] = pool_ref[pl.ds(0, NG), :] + ps
    pool_ref[pl.ds(2 * NG, NG), :] = pool_ref[pl.ds(2 * NG, NG), :] + pc

    @pl.when(i == pl.num_programs(0) - 1)
    def _():
        cnt = jnp.maximum(pool_ref[pl.ds(2 * NG, NG), :], 1.0)
        pool_ref[pl.ds(NG, NG), :] = pool_ref[pl.ds(0, NG), :] / cnt


@jax.jit
def _tc_post(part, dbc, oht, b, w):
    nb = NP // R
    return pl.pallas_call(
        _tc_post_body,
        grid=(nb,),
        in_specs=[
            pl.BlockSpec((2, R, D), lambda i: (0, i, 0)),
            pl.BlockSpec((R, D), lambda i: (i, 0)),
            pl.BlockSpec((NG, R), lambda i: (0, i)),
            pl.BlockSpec((1, D), lambda i: (0, 0)),
            pl.BlockSpec((D, D), lambda i: (0, 0)),
        ],
        out_specs=[
            pl.BlockSpec((R, D), lambda i: (i, 0)),
            pl.BlockSpec((3 * NG, D), lambda i: (0, 0)),
        ],
        out_shape=[
            jax.ShapeDtypeStruct((NP, D), _f32),
            jax.ShapeDtypeStruct((3 * NG, D), _f32),
        ],
    )(part, dbc, oht, b, w)


def kernel(x, edge_index, edge_attr, batch, pre1_W, pre1_b, pre2_W, pre2_b,
           conv0_W, conv0_b, conv1_W, conv1_b, conv2_W, conv2_b):
    row = edge_index[0]
    col = edge_index[1]
    ew = jnp.ravel(edge_attr).astype(_f32)

    loop_idx = jnp.arange(N_NODES, dtype=_i32)
    padn = EP - N_EDGES - N_NODES
    pad_idx = jnp.arange(padn, dtype=_i32) % N_NODES
    row3d = jnp.concatenate([row, loop_idx, pad_idx]).reshape(32, TR, 128)
    col3d = jnp.concatenate([col, loop_idx, pad_idx]).reshape(32, TR, 128)
    rc4d = jnp.stack([row3d, col3d], axis=2)
    ew3d = jnp.concatenate([ew, jnp.ones((N_NODES,), _f32),
                            jnp.zeros((padn,), _f32)]).reshape(32, TR, 128)
    x_p = jnp.pad(x, ((0, NP - N_NODES), (0, 0)))

    batch_p = jnp.concatenate([batch, jnp.full((NP - N_NODES,), NG, _i32)])
    oht = (batch_p[None, :] ==
           jnp.arange(NG, dtype=_i32)[:, None]).astype(_f32)

    dbc = _sc_prep(row3d, ew3d)
    gs = _tc_pre(x_p, dbc, pre1_W, pre1_b.reshape(1, D),
                 pre2_W, pre2_b.reshape(1, D), conv0_W)

    pools = []
    layer_b = (conv0_b, conv1_b, conv2_b)
    layer_wnext = (conv1_W, conv2_W, conv2_W)
    for l in range(3):
        part = _sc_agg(gs, rc4d, ew3d)
        gs, pool = _tc_post(part, dbc, oht, layer_b[l].reshape(1, D),
                            layer_wnext[l])
        pools.append(pool)

    return jnp.concatenate(
        [jnp.concatenate([p[:NG], p[NG:2 * NG]], axis=1) for p in pools],
        axis=1)
